# baseline (speedup 1.0000x reference)
import functools

import jax
import jax.numpy as jnp
from jax import lax
from jax.experimental import pallas as pl
from jax.experimental.pallas import tpu as pltpu
from jax.experimental.pallas import tpu_sc as plsc

N = 100000
NODE = 128
EDGE = 16
OUT = 128
E_TOT = 400000
DEGREES = (1, 2, 4, 8)
NPER = 25000

NW = 32
CHW = 784
LO_MAX = NPER - CHW
CH_D = {1: 112, 2: 112, 4: 56, 8: 24}
GBUF = 224
ABUF = 112


def _gather_slices(total):
    out = []
    off = 0
    while off < total:
        sz = min(128, total - off)
        out.append((off, sz))
        off += sz
    return out


def _sc_gather_sum(node_hbm, edge_hbm,
                   ni1, ei1, ni2, ei2, ni4, ei4, ni8, ei8,
                   sn_hbm, se_hbm,
                   idxn, idxe, g_node, g_edge, acc_node, acc_edge,
                   sem_gn, sem_ge, sem_sn, sem_se):
    nidx = {1: ni1, 2: ni2, 4: ni4, 8: ni8}
    eidx = {1: ei1, 2: ei2, 4: ei4, 8: ei8}
    wid = lax.axis_index("s") * 2 + lax.axis_index("c")
    lo = jnp.minimum(wid * CHW, LO_MAX)

    for di, d in enumerate(DEGREES):
        ch = CH_D[d]
        g = ch * d
        nch = -(-CHW // ch)
        out_off = di * NPER
        slices = _gather_slices(g)
        n_dst = acc_node if d == 1 else g_node
        e_dst = acc_edge if d == 1 else g_edge

        def chunk_base(k):
            return lo + jnp.minimum(k * ch, CHW - ch)

        def stage_and_fire(k, p):
            base = chunk_base(k)
            pltpu.sync_copy(nidx[d].at[pl.ds(base * d, g)],
                            idxn.at[p, pl.ds(0, g)])
            pltpu.sync_copy(eidx[d].at[pl.ds(base * d, g)],
                            idxe.at[p, pl.ds(0, g)])
            for off, sz in slices:
                pltpu.async_copy(node_hbm.at[idxn.at[p, pl.ds(off, sz)]],
                                 n_dst.at[p, pl.ds(off, sz)], sem_gn)
                pltpu.async_copy(edge_hbm.at[idxe.at[p, pl.ds(off, sz)]],
                                 e_dst.at[p, pl.ds(off, sz)], sem_ge)

        def drain_gathers(p):
            for off, sz in slices:
                pltpu.make_async_copy(node_hbm.at[pl.ds(0, sz)],
                                      n_dst.at[p, pl.ds(off, sz)], sem_gn).wait()
                pltpu.make_async_copy(edge_hbm.at[pl.ds(0, sz)],
                                      e_dst.at[p, pl.ds(off, sz)], sem_ge).wait()

        def drain_scatters(p):
            pltpu.make_async_copy(acc_node.at[p, pl.ds(0, ch)],
                                  sn_hbm.at[pl.ds(lo, ch)], sem_sn).wait()
            pltpu.make_async_copy(acc_edge.at[p, pl.ds(0, ch)],
                                  se_hbm.at[pl.ds(lo, ch)], sem_se).wait()

        stage_and_fire(0, 0)

        def chunk_body(k, _):
            p = lax.rem(k, 2)
            drain_gathers(p)

            @pl.when(k >= 2)
            def _():
                drain_scatters(p)

            if d > 1:
                def sum_body(b, _):
                    row = b * d
                    for cseg in range(NODE // 16):
                        cs = pl.ds(cseg * 16, 16)
                        v = g_node[p, row, cs]
                        for j in range(1, d):
                            v = v + g_node[p, row + j, cs]
                        acc_node[p, b, cs] = v
                    ev = g_edge[p, row, :]
                    for j in range(1, d):
                        ev = ev + g_edge[p, row + j, :]
                    acc_edge[p, b, :] = ev
                    return 0
                lax.fori_loop(0, ch, sum_body, 0, unroll=False)

            base = chunk_base(k)
            pltpu.async_copy(acc_node.at[p, pl.ds(0, ch)],
                             sn_hbm.at[pl.ds(out_off + base, ch)], sem_sn)
            pltpu.async_copy(acc_edge.at[p, pl.ds(0, ch)],
                             se_hbm.at[pl.ds(out_off + base, ch)], sem_se)

            @pl.when(k + 1 < nch)
            def _():
                stage_and_fire(k + 1, 1 - p)
            return 0

        lax.fori_loop(0, nch, chunk_body, 0, unroll=False)

        drain_scatters((nch - 1) % 2)
        drain_scatters((nch - 2) % 2)


def _run_sc_gather(node_repr, edge_repr, flat_idx):
    mesh = plsc.VectorSubcoreMesh(core_axis_name="c", subcore_axis_name="s")
    fn = functools.partial(
        pl.kernel,
        out_type=[
            jax.ShapeDtypeStruct((N, NODE), jnp.float32),
            jax.ShapeDtypeStruct((N, EDGE), jnp.float32),
        ],
        mesh=mesh,
        scratch_types=[
            pltpu.VMEM((2, GBUF), jnp.int32),
            pltpu.VMEM((2, GBUF), jnp.int32),
            pltpu.VMEM((2, GBUF, NODE), jnp.float32),
            pltpu.VMEM((2, GBUF, EDGE), jnp.float32),
            pltpu.VMEM((2, ABUF, NODE), jnp.float32),
            pltpu.VMEM((2, ABUF, EDGE), jnp.float32),
            pltpu.SemaphoreType.DMA,
            pltpu.SemaphoreType.DMA,
            pltpu.SemaphoreType.DMA,
            pltpu.SemaphoreType.DMA,
        ],
        compiler_params=pltpu.CompilerParams(use_tc_tiling_on_sc=False),
    )(_sc_gather_sum)
    return fn(node_repr, edge_repr, *flat_idx)


B_TC = 1000


def _dense_body(node_ref, sn_ref, se_ref, ws_ref, wn_ref, we_ref, bias_ref,
                y_ref, stats_ref):
    i = pl.program_id(0)
    y = jnp.dot(node_ref[...], ws_ref[...], preferred_element_type=jnp.float32)
    y += jnp.dot(sn_ref[...], wn_ref[0], preferred_element_type=jnp.float32)
    y += jnp.dot(se_ref[...], we_ref[0], preferred_element_type=jnp.float32)
    y += bias_ref[...]
    y_ref[...] = y

    @pl.when(i == 0)
    def _():
        stats_ref[...] = jnp.zeros_like(stats_ref)

    s1 = jnp.sum(y, axis=0, keepdims=True)
    s2 = jnp.sum(y * y, axis=0, keepdims=True)
    stats_ref[...] += jnp.concatenate([s1, s2], axis=0)


def _norm_body(y_ref, stats_ref, out_ref):
    s = stats_ref[...]
    mean = s[0:1] * (1.0 / N)
    var = s[1:2] * (1.0 / N) - mean * mean
    inv = lax.rsqrt(var + 1e-5)
    out_ref[...] = jnp.maximum((y_ref[...] - mean) * inv, 0.0)


def kernel(node_repr, edge_repr, node_idx_d1, edge_idx_d1, node_idx_d2,
           edge_idx_d2, node_idx_d4, edge_idx_d4, node_idx_d8, edge_idx_d8,
           W_self, W_d1, W_d2, W_d4, W_d8, bias):
    flat_idx = []
    for ni, ei in ((node_idx_d1, edge_idx_d1), (node_idx_d2, edge_idx_d2),
                   (node_idx_d4, edge_idx_d4), (node_idx_d8, edge_idx_d8)):
        flat_idx.append(ni.reshape(-1))
        flat_idx.append(ei.reshape(-1))

    sn, se = _run_sc_gather(node_repr, edge_repr, flat_idx)

    wn = jnp.stack([W_d1[:NODE], W_d2[:NODE], W_d4[:NODE], W_d8[:NODE]])
    we = jnp.stack([W_d1[NODE:], W_d2[NODE:], W_d4[NODE:], W_d8[NODE:]])

    nblocks = N // B_TC
    per_deg = NPER // B_TC
    y, stats = pl.pallas_call(
        _dense_body,
        grid=(nblocks,),
        in_specs=[
            pl.BlockSpec((B_TC, NODE), lambda i: (i, 0)),
            pl.BlockSpec((B_TC, NODE), lambda i: (i, 0)),
            pl.BlockSpec((B_TC, EDGE), lambda i: (i, 0)),
            pl.BlockSpec((NODE, OUT), lambda i: (0, 0)),
            pl.BlockSpec((1, NODE, OUT), lambda i: (i // per_deg, 0, 0)),
            pl.BlockSpec((1, EDGE, OUT), lambda i: (i // per_deg, 0, 0)),
            pl.BlockSpec((1, OUT), lambda i: (0, 0)),
        ],
        out_specs=[
            pl.BlockSpec((B_TC, OUT), lambda i: (i, 0)),
            pl.BlockSpec((2, OUT), lambda i: (0, 0)),
        ],
        out_shape=[
            jax.ShapeDtypeStruct((N, OUT), jnp.float32),
            jax.ShapeDtypeStruct((2, OUT), jnp.float32),
        ],
    )(node_repr, sn, se, W_self, wn, we, bias)

    out = pl.pallas_call(
        _norm_body,
        grid=(nblocks,),
        in_specs=[
            pl.BlockSpec((B_TC, OUT), lambda i: (i, 0)),
            pl.BlockSpec((2, OUT), lambda i: (0, 0)),
        ],
        out_specs=pl.BlockSpec((B_TC, OUT), lambda i: (i, 0)),
        out_shape=jax.ShapeDtypeStruct((N, OUT), jnp.float32),
    )(y, stats)
    return out


# speedup vs baseline: 1.0341x; 1.0341x over previous
import functools

import jax
import jax.numpy as jnp
from jax import lax
from jax.experimental import pallas as pl
from jax.experimental.pallas import tpu as pltpu
from jax.experimental.pallas import tpu_sc as plsc

N = 100000
NODE = 128
EDGE = 16
OUT = 128
E_TOT = 400000
DEGREES = (1, 2, 4, 8)
NPER = 25000

NW = 32
CHW = 784
LO_MAX = NPER - CHW
CH_D = {1: 256, 2: 112, 4: 56, 8: 32}
GBUF = 256
ABUF = 112


def _gather_slices(total):
    out = []
    off = 0
    while off < total:
        sz = min(128, total - off)
        out.append((off, sz))
        off += sz
    return out


def _sc_gather_sum(node_hbm, edge_hbm,
                   ni1, ei1, ni2, ei2, ni4, ei4, ni8, ei8,
                   sn_hbm, se_hbm,
                   i2n1, i2e1, i2n2, i2e2, i2n4, i2e4, i2n8, i2e8,
                   idxn, idxe, g_node, g_edge, acc_node, acc_edge,
                   sem_gn, sem_ge, sem_sn, sem_se):
    nidx = {1: ni1, 2: ni2, 4: ni4, 8: ni8}
    eidx = {1: ei1, 2: ei2, 4: ei4, 8: ei8}
    stg_n = {1: i2n1, 2: i2n2, 4: i2n4, 8: i2n8}
    stg_e = {1: i2e1, 2: i2e2, 4: i2e4, 8: i2e8}
    shift = {1: 0, 2: 1, 4: 2, 8: 3}
    wid = lax.axis_index("s") * 2 + lax.axis_index("c")
    lo = jnp.minimum(wid * CHW, LO_MAX)

    for di, d in enumerate(DEGREES):
        ch = CH_D[d]
        g = ch * d
        nch = -(-CHW // ch)
        out_off = di * NPER
        slices = _gather_slices(g)
        n_acc = g_node if d == 1 else acc_node
        e_acc = g_edge if d == 1 else acc_edge

        def chunk_base(k):
            return lo + jnp.minimum(k * ch, CHW - ch)

        def compact(stg, p, dst):
            pvec = jnp.full((16,), p, jnp.int32)
            for v in range(g // 16):
                l = lax.iota(jnp.int32, 16) + (v * 16)
                row = lax.shift_right_logical(l, shift[d])
                col = lax.bitwise_and(l, d - 1)
                vals = plsc.load_gather(stg, [pvec, row, col])
                dst[p, pl.ds(v * 16, 16)] = vals

        def stage_idx(k, p):
            base = chunk_base(k)
            pltpu.sync_copy(nidx[d].at[pl.ds(base, ch)], stg_n[d].at[p])
            pltpu.sync_copy(eidx[d].at[pl.ds(base, ch)], stg_e[d].at[p])
            compact(stg_n[d], p, idxn)
            compact(stg_e[d], p, idxe)

        def fire_gathers(p):
            for off, sz in slices:
                pltpu.async_copy(node_hbm.at[idxn.at[p, pl.ds(off, sz)]],
                                 g_node.at[p, pl.ds(off, sz)], sem_gn)
                pltpu.async_copy(edge_hbm.at[idxe.at[p, pl.ds(off, sz)]],
                                 g_edge.at[p, pl.ds(off, sz)], sem_ge)

        def drain_gathers(p):
            for off, sz in slices:
                pltpu.make_async_copy(node_hbm.at[pl.ds(0, sz)],
                                      g_node.at[p, pl.ds(off, sz)], sem_gn).wait()
                pltpu.make_async_copy(edge_hbm.at[pl.ds(0, sz)],
                                      g_edge.at[p, pl.ds(off, sz)], sem_ge).wait()

        def drain_scatters(p):
            pltpu.make_async_copy(n_acc.at[p, pl.ds(0, ch)],
                                  sn_hbm.at[pl.ds(lo, ch)], sem_sn).wait()
            pltpu.make_async_copy(e_acc.at[p, pl.ds(0, ch)],
                                  se_hbm.at[pl.ds(lo, ch)], sem_se).wait()

        stage_idx(0, 0)
        fire_gathers(0)

        def chunk_body(k, _):
            p = lax.rem(k, 2)

            @pl.when(k + 1 < nch)
            def _():
                stage_idx(k + 1, 1 - p)

            drain_gathers(p)

            if d > 1:
                @pl.when(k >= 2)
                def _():
                    drain_scatters(p)

                def sum_body(b, _):
                    row = b * d
                    for cseg in range(NODE // 16):
                        cs = pl.ds(cseg * 16, 16)
                        v = g_node[p, row, cs]
                        for j in range(1, d):
                            v = v + g_node[p, row + j, cs]
                        acc_node[p, b, cs] = v
                    ev = g_edge[p, row, :]
                    for j in range(1, d):
                        ev = ev + g_edge[p, row + j, :]
                    acc_edge[p, b, :] = ev
                    return 0
                lax.fori_loop(0, ch, sum_body, 0, unroll=False)

            base = chunk_base(k)
            pltpu.async_copy(n_acc.at[p, pl.ds(0, ch)],
                             sn_hbm.at[pl.ds(out_off + base, ch)], sem_sn)
            pltpu.async_copy(e_acc.at[p, pl.ds(0, ch)],
                             se_hbm.at[pl.ds(out_off + base, ch)], sem_se)

            @pl.when(k + 1 < nch)
            def _():
                if d == 1:
                    @pl.when(k >= 1)
                    def _():
                        drain_scatters(1 - p)
                fire_gathers(1 - p)
            return 0

        lax.fori_loop(0, nch, chunk_body, 0, unroll=False)

        drain_scatters((nch - 1) % 2)
        drain_scatters((nch - 2) % 2)


def _run_sc_gather(node_repr, edge_repr, flat_idx):
    mesh = plsc.VectorSubcoreMesh(core_axis_name="c", subcore_axis_name="s")
    fn = functools.partial(
        pl.kernel,
        out_type=[
            jax.ShapeDtypeStruct((N, NODE), jnp.float32),
            jax.ShapeDtypeStruct((N, EDGE), jnp.float32),
        ],
        mesh=mesh,
        scratch_types=(
            [pltpu.VMEM((2, CH_D[d], d), jnp.int32)
             for d in DEGREES for _ in range(2)]
            + [
                pltpu.VMEM((2, GBUF), jnp.int32),
                pltpu.VMEM((2, GBUF), jnp.int32),
                pltpu.VMEM((2, GBUF, NODE), jnp.float32),
                pltpu.VMEM((2, GBUF, EDGE), jnp.float32),
                pltpu.VMEM((2, ABUF, NODE), jnp.float32),
                pltpu.VMEM((2, ABUF, EDGE), jnp.float32),
                pltpu.SemaphoreType.DMA,
                pltpu.SemaphoreType.DMA,
                pltpu.SemaphoreType.DMA,
                pltpu.SemaphoreType.DMA,
            ]
        ),
        compiler_params=pltpu.CompilerParams(use_tc_tiling_on_sc=False,
                                             needs_layout_passes=False),
    )(_sc_gather_sum)
    return fn(node_repr, edge_repr, *flat_idx)


B_TC = 5000


def _dense_body(node_ref, sn_ref, se_ref, ws_ref, wn_ref, we_ref, bias_ref,
                y_ref, stats_ref):
    i = pl.program_id(0)
    y = jnp.dot(node_ref[...], ws_ref[...], preferred_element_type=jnp.float32)
    y += jnp.dot(sn_ref[...], wn_ref[0], preferred_element_type=jnp.float32)
    y += jnp.dot(se_ref[...], we_ref[0], preferred_element_type=jnp.float32)
    y += bias_ref[...]
    y_ref[...] = y

    @pl.when(i == 0)
    def _():
        stats_ref[...] = jnp.zeros_like(stats_ref)

    s1 = jnp.sum(y, axis=0, keepdims=True)
    s2 = jnp.sum(y * y, axis=0, keepdims=True)
    stats_ref[...] += jnp.concatenate([s1, s2], axis=0)


def _norm_body(y_ref, stats_ref, out_ref):
    s = stats_ref[...]
    mean = s[0:1] * (1.0 / N)
    var = s[1:2] * (1.0 / N) - mean * mean
    inv = lax.rsqrt(var + 1e-5)
    out_ref[...] = jnp.maximum((y_ref[...] - mean) * inv, 0.0)


def kernel(node_repr, edge_repr, node_idx_d1, edge_idx_d1, node_idx_d2,
           edge_idx_d2, node_idx_d4, edge_idx_d4, node_idx_d8, edge_idx_d8,
           W_self, W_d1, W_d2, W_d4, W_d8, bias):
    idx_2d = [node_idx_d1, edge_idx_d1, node_idx_d2, edge_idx_d2,
              node_idx_d4, edge_idx_d4, node_idx_d8, edge_idx_d8]

    sn, se = _run_sc_gather(node_repr, edge_repr, idx_2d)

    wn = jnp.stack([W_d1[:NODE], W_d2[:NODE], W_d4[:NODE], W_d8[:NODE]])
    we = jnp.stack([W_d1[NODE:], W_d2[NODE:], W_d4[NODE:], W_d8[NODE:]])

    nblocks = N // B_TC
    per_deg = NPER // B_TC
    y, stats = pl.pallas_call(
        _dense_body,
        grid=(nblocks,),
        in_specs=[
            pl.BlockSpec((B_TC, NODE), lambda i: (i, 0)),
            pl.BlockSpec((B_TC, NODE), lambda i: (i, 0)),
            pl.BlockSpec((B_TC, EDGE), lambda i: (i, 0)),
            pl.BlockSpec((NODE, OUT), lambda i: (0, 0)),
            pl.BlockSpec((1, NODE, OUT), lambda i: (i // per_deg, 0, 0)),
            pl.BlockSpec((1, EDGE, OUT), lambda i: (i // per_deg, 0, 0)),
            pl.BlockSpec((1, OUT), lambda i: (0, 0)),
        ],
        out_specs=[
            pl.BlockSpec((B_TC, OUT), lambda i: (i, 0)),
            pl.BlockSpec((2, OUT), lambda i: (0, 0)),
        ],
        out_shape=[
            jax.ShapeDtypeStruct((N, OUT), jnp.float32),
            jax.ShapeDtypeStruct((2, OUT), jnp.float32),
        ],
    )(node_repr, sn, se, W_self, wn, we, bias)

    out = pl.pallas_call(
        _norm_body,
        grid=(nblocks,),
        in_specs=[
            pl.BlockSpec((B_TC, OUT), lambda i: (i, 0)),
            pl.BlockSpec((2, OUT), lambda i: (0, 0)),
        ],
        out_specs=pl.BlockSpec((B_TC, OUT), lambda i: (i, 0)),
        out_shape=jax.ShapeDtypeStruct((N, OUT), jnp.float32),
    )(y, stats)
    return out


# speedup vs baseline: 1.2817x; 1.2394x over previous
import functools

import jax
import jax.numpy as jnp
from jax import lax
from jax.experimental import pallas as pl
from jax.experimental.pallas import tpu as pltpu
from jax.experimental.pallas import tpu_sc as plsc

N = 100000
NODE = 128
EDGE = 16
OUT = 128
E_TOT = 400000
DEGREES = (1, 2, 4, 8)
NPER = 25000

NW = 32
CHW = 784
LO_MAX = NPER - CHW
CH_D = {1: 256, 2: 112, 4: 56, 8: 32}
GBUF = 256
ABUF = 112


def _idx_slices(ch):
    out = []
    off = 0
    while off < ch:
        sz = min(128, ch - off)
        out.append((off, sz))
        off += sz
    return out


def _sc_gather_sum(node_hbm, edge_hbm,
                   ni1, ei1, ni2, ei2, ni4, ei4, ni8, ei8,
                   sn_hbm, se_hbm,
                   sgn1, sge1, sgn2, sge2, sgn4, sge4, sgn8, sge8,
                   g_node, g_edge, acc_node, acc_edge,
                   sem_gn, sem_ge, sem_sn, sem_se):
    nidx = {1: ni1, 2: ni2, 4: ni4, 8: ni8}
    eidx = {1: ei1, 2: ei2, 4: ei4, 8: ei8}
    stg_n = {1: sgn1, 2: sgn2, 4: sgn4, 8: sgn8}
    stg_e = {1: sge1, 2: sge2, 4: sge4, 8: sge8}
    wid = lax.axis_index("s") * 2 + lax.axis_index("c")
    lo = jnp.minimum(wid * CHW, LO_MAX)

    for di, d in enumerate(DEGREES):
        ch = CH_D[d]
        nch = -(-CHW // ch)
        out_off = di * NPER
        slices = _idx_slices(ch)
        n_acc = g_node if d == 1 else acc_node
        e_acc = g_edge if d == 1 else acc_edge

        def chunk_base(k, ch=ch):
            return lo + jnp.minimum(k * ch, CHW - ch)

        def stage_idx(k, p, d=d, ch=ch):
            base = chunk_base(k, ch)
            pltpu.sync_copy(nidx[d].at[pl.ds(0, d), pl.ds(base, ch)],
                            stg_n[d].at[p])
            pltpu.sync_copy(eidx[d].at[pl.ds(0, d), pl.ds(base, ch)],
                            stg_e[d].at[p])

        def fire_gathers(p, d=d, ch=ch, slices=slices):
            for j in range(d):
                for off, sz in slices:
                    dst = pl.ds(j * ch + off, sz)
                    pltpu.async_copy(
                        node_hbm.at[stg_n[d].at[p, j, pl.ds(off, sz)]],
                        g_node.at[p, dst], sem_gn)
                    pltpu.async_copy(
                        edge_hbm.at[stg_e[d].at[p, j, pl.ds(off, sz)]],
                        g_edge.at[p, dst], sem_ge)

        def drain_gathers(p, d=d, ch=ch, slices=slices):
            for j in range(d):
                for off, sz in slices:
                    dst = pl.ds(j * ch + off, sz)
                    pltpu.make_async_copy(node_hbm.at[pl.ds(0, sz)],
                                          g_node.at[p, dst], sem_gn).wait()
                    pltpu.make_async_copy(edge_hbm.at[pl.ds(0, sz)],
                                          g_edge.at[p, dst], sem_ge).wait()

        def drain_scatters(p, ch=ch, n_acc=n_acc, e_acc=e_acc):
            pltpu.make_async_copy(n_acc.at[p, pl.ds(0, ch)],
                                  sn_hbm.at[pl.ds(lo, ch)], sem_sn).wait()
            pltpu.make_async_copy(e_acc.at[p, pl.ds(0, ch)],
                                  se_hbm.at[pl.ds(lo, ch)], sem_se).wait()

        stage_idx(0, 0)
        fire_gathers(0)

        def chunk_body(k, _, d=d, ch=ch, nch=nch, out_off=out_off,
                       n_acc=n_acc, e_acc=e_acc):
            p = lax.rem(k, 2)

            @pl.when(k + 1 < nch)
            def _():
                stage_idx(k + 1, 1 - p)

            drain_gathers(p)

            if d > 1:
                @pl.when(k >= 2)
                def _():
                    drain_scatters(p)

                def sum_body(b, _):
                    for cseg in range(NODE // 16):
                        cs = pl.ds(cseg * 16, 16)
                        v = g_node[p, b, cs]
                        for j in range(1, d):
                            v = v + g_node[p, j * ch + b, cs]
                        acc_node[p, b, cs] = v
                    ev = g_edge[p, b, :]
                    for j in range(1, d):
                        ev = ev + g_edge[p, j * ch + b, :]
                    acc_edge[p, b, :] = ev
                    return 0
                lax.fori_loop(0, ch, sum_body, 0, unroll=False)

            base = chunk_base(k, ch)
            pltpu.async_copy(n_acc.at[p, pl.ds(0, ch)],
                             sn_hbm.at[pl.ds(out_off + base, ch)], sem_sn)
            pltpu.async_copy(e_acc.at[p, pl.ds(0, ch)],
                             se_hbm.at[pl.ds(out_off + base, ch)], sem_se)

            @pl.when(k + 1 < nch)
            def _():
                if d == 1:
                    @pl.when(k >= 1)
                    def _():
                        drain_scatters(1 - p)
                fire_gathers(1 - p)
            return 0

        lax.fori_loop(0, nch, chunk_body, 0, unroll=False)

        drain_scatters((nch - 1) % 2)
        drain_scatters((nch - 2) % 2)


def _run_sc_gather(node_repr, edge_repr, idx_t):
    mesh = plsc.VectorSubcoreMesh(core_axis_name="c", subcore_axis_name="s")
    fn = functools.partial(
        pl.kernel,
        out_type=[
            jax.ShapeDtypeStruct((N, NODE), jnp.float32),
            jax.ShapeDtypeStruct((N, EDGE), jnp.float32),
        ],
        mesh=mesh,
        scratch_types=(
            [pltpu.VMEM((2, d, CH_D[d]), jnp.int32)
             for d in DEGREES for _ in range(2)]
            + [
                pltpu.VMEM((2, GBUF, NODE), jnp.float32),
                pltpu.VMEM((2, GBUF, EDGE), jnp.float32),
                pltpu.VMEM((2, ABUF, NODE), jnp.float32),
                pltpu.VMEM((2, ABUF, EDGE), jnp.float32),
                pltpu.SemaphoreType.DMA,
                pltpu.SemaphoreType.DMA,
                pltpu.SemaphoreType.DMA,
                pltpu.SemaphoreType.DMA,
            ]
        ),
        compiler_params=pltpu.CompilerParams(use_tc_tiling_on_sc=False),
    )(_sc_gather_sum)
    return fn(node_repr, edge_repr, *idx_t)


B_TC = 5000


def _dense_body(node_ref, sn_ref, se_ref, ws_ref, w1_ref, w2_ref, w4_ref,
                w8_ref, bias_ref, y_ref, stats_ref):
    i = pl.program_id(0)
    di = i // (NPER // B_TC)
    w = jnp.where(di == 0, w1_ref[...],
                  jnp.where(di == 1, w2_ref[...],
                            jnp.where(di == 2, w4_ref[...], w8_ref[...])))
    y = jnp.dot(node_ref[...], ws_ref[...], preferred_element_type=jnp.float32)
    y += jnp.dot(sn_ref[...], w[:NODE], preferred_element_type=jnp.float32)
    y += jnp.dot(se_ref[...], w[NODE:], preferred_element_type=jnp.float32)
    y += bias_ref[...]
    y_ref[...] = y

    @pl.when(i == 0)
    def _():
        stats_ref[...] = jnp.zeros_like(stats_ref)

    s1 = jnp.sum(y, axis=0, keepdims=True)
    s2 = jnp.sum(y * y, axis=0, keepdims=True)
    stats_ref[...] += jnp.concatenate([s1, s2], axis=0)


def _norm_body(y_ref, stats_ref, out_ref):
    s = stats_ref[...]
    mean = s[0:1] * (1.0 / N)
    var = s[1:2] * (1.0 / N) - mean * mean
    inv = lax.rsqrt(var + 1e-5)
    out_ref[...] = jnp.maximum((y_ref[...] - mean) * inv, 0.0)


def kernel(node_repr, edge_repr, node_idx_d1, edge_idx_d1, node_idx_d2,
           edge_idx_d2, node_idx_d4, edge_idx_d4, node_idx_d8, edge_idx_d8,
           W_self, W_d1, W_d2, W_d4, W_d8, bias):
    idx_t = [jnp.transpose(a) for a in
             (node_idx_d1, edge_idx_d1, node_idx_d2, edge_idx_d2,
              node_idx_d4, edge_idx_d4, node_idx_d8, edge_idx_d8)]

    sn, se = _run_sc_gather(node_repr, edge_repr, idx_t)

    nblocks = N // B_TC
    y, stats = pl.pallas_call(
        _dense_body,
        grid=(nblocks,),
        in_specs=[
            pl.BlockSpec((B_TC, NODE), lambda i: (i, 0)),
            pl.BlockSpec((B_TC, NODE), lambda i: (i, 0)),
            pl.BlockSpec((B_TC, EDGE), lambda i: (i, 0)),
            pl.BlockSpec((NODE, OUT), lambda i: (0, 0)),
            pl.BlockSpec((NODE + EDGE, OUT), lambda i: (0, 0)),
            pl.BlockSpec((NODE + EDGE, OUT), lambda i: (0, 0)),
            pl.BlockSpec((NODE + EDGE, OUT), lambda i: (0, 0)),
            pl.BlockSpec((NODE + EDGE, OUT), lambda i: (0, 0)),
            pl.BlockSpec((1, OUT), lambda i: (0, 0)),
        ],
        out_specs=[
            pl.BlockSpec((B_TC, OUT), lambda i: (i, 0)),
            pl.BlockSpec((2, OUT), lambda i: (0, 0)),
        ],
        out_shape=[
            jax.ShapeDtypeStruct((N, OUT), jnp.float32),
            jax.ShapeDtypeStruct((2, OUT), jnp.float32),
        ],
    )(node_repr, sn, se, W_self, W_d1, W_d2, W_d4, W_d8, bias)

    out = pl.pallas_call(
        _norm_body,
        grid=(nblocks,),
        in_specs=[
            pl.BlockSpec((B_TC, OUT), lambda i: (i, 0)),
            pl.BlockSpec((2, OUT), lambda i: (0, 0)),
        ],
        out_specs=pl.BlockSpec((B_TC, OUT), lambda i: (i, 0)),
        out_shape=jax.ShapeDtypeStruct((N, OUT), jnp.float32),
    )(y, stats)
    return out


# speedup vs baseline: 1.4103x; 1.1003x over previous
import functools

import jax
import jax.numpy as jnp
from jax import lax
from jax.experimental import pallas as pl
from jax.experimental.pallas import tpu as pltpu
from jax.experimental.pallas import tpu_sc as plsc

N = 100000
NODE = 128
EDGE = 16
OUT = 128
E_TOT = 400000
DEGREES = (1, 2, 4, 8)
NPER = 25000

NW = 32
CHW = 784
LO_MAX = NPER - CHW
CH_D = {1: 256, 2: 112, 4: 56, 8: 32}
GBUF = 256
ABUF = 112


def _idx_slices(ch):
    out = []
    off = 0
    while off < ch:
        sz = min(128, ch - off)
        out.append((off, sz))
        off += sz
    return out


def _sc_gather_sum(node_hbm, edge_hbm,
                   ni1, ei1, ni2, ei2, ni4, ei4, ni8, ei8,
                   sn_hbm, se_hbm,
                   sgn1, sge1, sgn2, sge2, sgn4, sge4, sgn8, sge8,
                   g_node, g_edge, acc_node, acc_edge,
                   sem_gn, sem_ge, sem_sn, sem_se):
    nidx = {1: ni1, 2: ni2, 4: ni4, 8: ni8}
    eidx = {1: ei1, 2: ei2, 4: ei4, 8: ei8}
    stg_n = {1: sgn1, 2: sgn2, 4: sgn4, 8: sgn8}
    stg_e = {1: sge1, 2: sge2, 4: sge4, 8: sge8}
    wid = lax.axis_index("s") * 2 + lax.axis_index("c")
    lo = jnp.minimum(wid * CHW, LO_MAX)

    for di, d in enumerate(DEGREES):
        ch = CH_D[d]
        nch = -(-CHW // ch)
        out_off = di * NPER
        slices = _idx_slices(ch)
        n_acc = g_node if d == 1 else acc_node
        e_acc = g_edge if d == 1 else acc_edge

        def chunk_base(k, ch=ch):
            return lo + jnp.minimum(k * ch, CHW - ch)

        def stage_idx(k, p, d=d, ch=ch):
            base = chunk_base(k, ch)
            pltpu.sync_copy(nidx[d].at[pl.ds(0, d), pl.ds(base, ch)],
                            stg_n[d].at[p])
            pltpu.sync_copy(eidx[d].at[pl.ds(0, d), pl.ds(base, ch)],
                            stg_e[d].at[p])

        def fire_gathers(p, d=d, ch=ch, slices=slices):
            for j in range(d):
                for off, sz in slices:
                    dst = pl.ds(j * ch + off, sz)
                    pltpu.async_copy(
                        node_hbm.at[stg_n[d].at[p, j, pl.ds(off, sz)]],
                        g_node.at[p, dst], sem_gn)
                    pltpu.async_copy(
                        edge_hbm.at[stg_e[d].at[p, j, pl.ds(off, sz)]],
                        g_edge.at[p, dst], sem_ge)

        def drain_gathers(p, d=d, ch=ch, slices=slices):
            for j in range(d):
                for off, sz in slices:
                    dst = pl.ds(j * ch + off, sz)
                    pltpu.make_async_copy(node_hbm.at[pl.ds(0, sz)],
                                          g_node.at[p, dst], sem_gn).wait()
                    pltpu.make_async_copy(edge_hbm.at[pl.ds(0, sz)],
                                          g_edge.at[p, dst], sem_ge).wait()

        def drain_scatters(p, ch=ch, n_acc=n_acc, e_acc=e_acc):
            pltpu.make_async_copy(n_acc.at[p, pl.ds(0, ch)],
                                  sn_hbm.at[pl.ds(lo, ch)], sem_sn).wait()
            pltpu.make_async_copy(e_acc.at[p, pl.ds(0, ch)],
                                  se_hbm.at[pl.ds(lo, ch)], sem_se).wait()

        stage_idx(0, 0)
        fire_gathers(0)

        def chunk_body(k, _, d=d, ch=ch, nch=nch, out_off=out_off,
                       n_acc=n_acc, e_acc=e_acc):
            p = lax.rem(k, 2)

            @pl.when(k + 1 < nch)
            def _():
                stage_idx(k + 1, 1 - p)

            drain_gathers(p)

            @pl.when(k + 1 < nch)
            def _():
                if d == 1:
                    @pl.when(k >= 1)
                    def _():
                        drain_scatters(1 - p)
                fire_gathers(1 - p)

            if d > 1:
                @pl.when(k >= 2)
                def _():
                    drain_scatters(p)

                def sum_body(b, _):
                    for cseg in range(NODE // 16):
                        cs = pl.ds(cseg * 16, 16)
                        v = g_node[p, b, cs]
                        for j in range(1, d):
                            v = v + g_node[p, j * ch + b, cs]
                        acc_node[p, b, cs] = v
                    ev = g_edge[p, b, :]
                    for j in range(1, d):
                        ev = ev + g_edge[p, j * ch + b, :]
                    acc_edge[p, b, :] = ev
                    return 0
                lax.fori_loop(0, ch, sum_body, 0, unroll=False)

            base = chunk_base(k, ch)
            pltpu.async_copy(n_acc.at[p, pl.ds(0, ch)],
                             sn_hbm.at[pl.ds(out_off + base, ch)], sem_sn)
            pltpu.async_copy(e_acc.at[p, pl.ds(0, ch)],
                             se_hbm.at[pl.ds(out_off + base, ch)], sem_se)
            return 0

        lax.fori_loop(0, nch, chunk_body, 0, unroll=False)

        drain_scatters((nch - 1) % 2)
        drain_scatters((nch - 2) % 2)


def _run_sc_gather(node_repr, edge_repr, idx_t):
    mesh = plsc.VectorSubcoreMesh(core_axis_name="c", subcore_axis_name="s")
    fn = functools.partial(
        pl.kernel,
        out_type=[
            jax.ShapeDtypeStruct((N, NODE), jnp.float32),
            jax.ShapeDtypeStruct((N, EDGE), jnp.float32),
        ],
        mesh=mesh,
        scratch_types=(
            [pltpu.VMEM((2, d, CH_D[d]), jnp.int32)
             for d in DEGREES for _ in range(2)]
            + [
                pltpu.VMEM((2, GBUF, NODE), jnp.float32),
                pltpu.VMEM((2, GBUF, EDGE), jnp.float32),
                pltpu.VMEM((2, ABUF, NODE), jnp.float32),
                pltpu.VMEM((2, ABUF, EDGE), jnp.float32),
                pltpu.SemaphoreType.DMA,
                pltpu.SemaphoreType.DMA,
                pltpu.SemaphoreType.DMA,
                pltpu.SemaphoreType.DMA,
            ]
        ),
        compiler_params=pltpu.CompilerParams(use_tc_tiling_on_sc=False),
    )(_sc_gather_sum)
    return fn(node_repr, edge_repr, *idx_t)


B_TC = 5000


def _dense_body(node_ref, sn_ref, se_ref, ws_ref, w1_ref, w2_ref, w4_ref,
                w8_ref, bias_ref, y_ref, stats_ref):
    i = pl.program_id(0)
    di = i // (NPER // B_TC)
    w = jnp.where(di == 0, w1_ref[...],
                  jnp.where(di == 1, w2_ref[...],
                            jnp.where(di == 2, w4_ref[...], w8_ref[...])))
    y = jnp.dot(node_ref[...], ws_ref[...], preferred_element_type=jnp.float32)
    y += jnp.dot(sn_ref[...], w[:NODE], preferred_element_type=jnp.float32)
    y += jnp.dot(se_ref[...], w[NODE:], preferred_element_type=jnp.float32)
    y += bias_ref[...]
    y_ref[...] = y

    @pl.when(i == 0)
    def _():
        stats_ref[...] = jnp.zeros_like(stats_ref)

    s1 = jnp.sum(y, axis=0, keepdims=True)
    s2 = jnp.sum(y * y, axis=0, keepdims=True)
    stats_ref[...] += jnp.concatenate([s1, s2], axis=0)


def _norm_body(y_ref, stats_ref, out_ref):
    s = stats_ref[...]
    mean = s[0:1] * (1.0 / N)
    var = s[1:2] * (1.0 / N) - mean * mean
    inv = lax.rsqrt(var + 1e-5)
    out_ref[...] = jnp.maximum((y_ref[...] - mean) * inv, 0.0)


def kernel(node_repr, edge_repr, node_idx_d1, edge_idx_d1, node_idx_d2,
           edge_idx_d2, node_idx_d4, edge_idx_d4, node_idx_d8, edge_idx_d8,
           W_self, W_d1, W_d2, W_d4, W_d8, bias):
    idx_t = [jnp.transpose(a) for a in
             (node_idx_d1, edge_idx_d1, node_idx_d2, edge_idx_d2,
              node_idx_d4, edge_idx_d4, node_idx_d8, edge_idx_d8)]

    sn, se = _run_sc_gather(node_repr, edge_repr, idx_t)

    nblocks = N // B_TC
    y, stats = pl.pallas_call(
        _dense_body,
        grid=(nblocks,),
        in_specs=[
            pl.BlockSpec((B_TC, NODE), lambda i: (i, 0)),
            pl.BlockSpec((B_TC, NODE), lambda i: (i, 0)),
            pl.BlockSpec((B_TC, EDGE), lambda i: (i, 0)),
            pl.BlockSpec((NODE, OUT), lambda i: (0, 0)),
            pl.BlockSpec((NODE + EDGE, OUT), lambda i: (0, 0)),
            pl.BlockSpec((NODE + EDGE, OUT), lambda i: (0, 0)),
            pl.BlockSpec((NODE + EDGE, OUT), lambda i: (0, 0)),
            pl.BlockSpec((NODE + EDGE, OUT), lambda i: (0, 0)),
            pl.BlockSpec((1, OUT), lambda i: (0, 0)),
        ],
        out_specs=[
            pl.BlockSpec((B_TC, OUT), lambda i: (i, 0)),
            pl.BlockSpec((2, OUT), lambda i: (0, 0)),
        ],
        out_shape=[
            jax.ShapeDtypeStruct((N, OUT), jnp.float32),
            jax.ShapeDtypeStruct((2, OUT), jnp.float32),
        ],
    )(node_repr, sn, se, W_self, W_d1, W_d2, W_d4, W_d8, bias)

    out = pl.pallas_call(
        _norm_body,
        grid=(nblocks,),
        in_specs=[
            pl.BlockSpec((B_TC, OUT), lambda i: (i, 0)),
            pl.BlockSpec((2, OUT), lambda i: (0, 0)),
        ],
        out_specs=pl.BlockSpec((B_TC, OUT), lambda i: (i, 0)),
        out_shape=jax.ShapeDtypeStruct((N, OUT), jnp.float32),
    )(y, stats)
    return out


# speedup vs baseline: 1.5358x; 1.0890x over previous
import functools

import jax
import jax.numpy as jnp
from jax import lax
from jax.experimental import pallas as pl
from jax.experimental.pallas import tpu as pltpu
from jax.experimental.pallas import tpu_sc as plsc

N = 100000
NODE = 128
EDGE = 16
OUT = 128
E_TOT = 400000
DEGREES = (1, 2, 4, 8)
NPER = 25000

NW = 32
CHW = 784
LO_MAX = NPER - CHW
CH_D = {1: 256, 2: 112, 4: 56, 8: 32}
GBUF = 256
ABUF = 112


def _idx_slices(ch):
    out = []
    off = 0
    while off < ch:
        sz = min(128, ch - off)
        out.append((off, sz))
        off += sz
    return out


def _make_sc_body(degrees):
    def body(*refs):
        it = iter(refs)
        node_hbm = next(it)
        edge_hbm = next(it)
        nidx, eidx = {}, {}
        for d in degrees:
            nidx[d] = next(it)
            eidx[d] = next(it)
        if degrees == (8,):
            next(it)
        sn_hbm = next(it)
        se_hbm = next(it)
        stg_n, stg_e = {}, {}
        for d in degrees:
            stg_n[d] = next(it)
            stg_e[d] = next(it)
        g_node = next(it)
        g_edge = next(it)
        acc_node = next(it)
        acc_edge = next(it)
        sem_gn = next(it)
        sem_ge = next(it)
        sem_sn = next(it)
        sem_se = next(it)

        wid = lax.axis_index("s") * 2 + lax.axis_index("c")
        lo = jnp.minimum(wid * CHW, LO_MAX)

        for di, d in enumerate(degrees):
            ch = CH_D[d]
            nch = -(-CHW // ch)
            out_off = di * NPER
            slices = _idx_slices(ch)
            n_acc = g_node if d == 1 else acc_node
            e_acc = g_edge if d == 1 else acc_edge

            def chunk_base(k, ch=ch):
                return lo + jnp.minimum(k * ch, CHW - ch)

            def stage_idx(k, p, d=d, ch=ch):
                base = chunk_base(k, ch)
                pltpu.sync_copy(nidx[d].at[pl.ds(0, d), pl.ds(base, ch)],
                                stg_n[d].at[p])
                pltpu.sync_copy(eidx[d].at[pl.ds(0, d), pl.ds(base, ch)],
                                stg_e[d].at[p])

            def fire_gathers(p, d=d, ch=ch, slices=slices):
                for j in range(d):
                    for off, sz in slices:
                        dst = pl.ds(j * ch + off, sz)
                        pltpu.async_copy(
                            node_hbm.at[stg_n[d].at[p, j, pl.ds(off, sz)]],
                            g_node.at[p, dst], sem_gn)
                        pltpu.async_copy(
                            edge_hbm.at[stg_e[d].at[p, j, pl.ds(off, sz)]],
                            g_edge.at[p, dst], sem_ge)

            def drain_gathers(p, d=d, ch=ch, slices=slices):
                for j in range(d):
                    for off, sz in slices:
                        dst = pl.ds(j * ch + off, sz)
                        pltpu.make_async_copy(node_hbm.at[pl.ds(0, sz)],
                                              g_node.at[p, dst], sem_gn).wait()
                        pltpu.make_async_copy(edge_hbm.at[pl.ds(0, sz)],
                                              g_edge.at[p, dst], sem_ge).wait()

            def drain_scatters(p, ch=ch, n_acc=n_acc, e_acc=e_acc):
                pltpu.make_async_copy(n_acc.at[p, pl.ds(0, ch)],
                                      sn_hbm.at[pl.ds(lo, ch)], sem_sn).wait()
                pltpu.make_async_copy(e_acc.at[p, pl.ds(0, ch)],
                                      se_hbm.at[pl.ds(lo, ch)], sem_se).wait()

            stage_idx(0, 0)
            fire_gathers(0)

            def chunk_body(k, _, d=d, ch=ch, nch=nch, out_off=out_off,
                           n_acc=n_acc, e_acc=e_acc, stage_idx=stage_idx,
                           fire_gathers=fire_gathers,
                           drain_gathers=drain_gathers,
                           drain_scatters=drain_scatters,
                           chunk_base=chunk_base):
                p = lax.rem(k, 2)

                @pl.when(k + 1 < nch)
                def _():
                    stage_idx(k + 1, 1 - p)

                drain_gathers(p)

                @pl.when(k + 1 < nch)
                def _():
                    if d == 1:
                        @pl.when(k >= 1)
                        def _():
                            drain_scatters(1 - p)
                    fire_gathers(1 - p)

                if d > 1:
                    @pl.when(k >= 2)
                    def _():
                        drain_scatters(p)

                    def sum_body(b, _):
                        for cseg in range(NODE // 16):
                            cs = pl.ds(cseg * 16, 16)
                            v = g_node[p, b, cs]
                            for j in range(1, d):
                                v = v + g_node[p, j * ch + b, cs]
                            acc_node[p, b, cs] = v
                        ev = g_edge[p, b, :]
                        for j in range(1, d):
                            ev = ev + g_edge[p, j * ch + b, :]
                        acc_edge[p, b, :] = ev
                        return 0
                    lax.fori_loop(0, ch, sum_body, 0, unroll=False)

                base = chunk_base(k)
                pltpu.async_copy(n_acc.at[p, pl.ds(0, ch)],
                                 sn_hbm.at[pl.ds(out_off + base, ch)], sem_sn)
                pltpu.async_copy(e_acc.at[p, pl.ds(0, ch)],
                                 se_hbm.at[pl.ds(out_off + base, ch)], sem_se)
                return 0

            lax.fori_loop(0, nch, chunk_body, 0, unroll=False)

            drain_scatters((nch - 1) % 2)
            drain_scatters((nch - 2) % 2)

    return body


def _run_sc_gather(node_repr, edge_repr, idx_t, degrees, extra=None):
    nrows = NPER * len(degrees)
    mesh = plsc.VectorSubcoreMesh(core_axis_name="c", subcore_axis_name="s")
    fn = functools.partial(
        pl.kernel,
        out_type=[
            jax.ShapeDtypeStruct((nrows, NODE), jnp.float32),
            jax.ShapeDtypeStruct((nrows, EDGE), jnp.float32),
        ],
        mesh=mesh,
        scratch_types=(
            [pltpu.VMEM((2, d, CH_D[d]), jnp.int32)
             for d in degrees for _ in range(2)]
            + [
                pltpu.VMEM((2, GBUF, NODE), jnp.float32),
                pltpu.VMEM((2, GBUF, EDGE), jnp.float32),
                pltpu.VMEM((2, ABUF, NODE), jnp.float32),
                pltpu.VMEM((2, ABUF, EDGE), jnp.float32),
                pltpu.SemaphoreType.DMA,
                pltpu.SemaphoreType.DMA,
                pltpu.SemaphoreType.DMA,
                pltpu.SemaphoreType.DMA,
            ]
        ),
        compiler_params=pltpu.CompilerParams(use_tc_tiling_on_sc=False),
    )(_make_sc_body(tuple(degrees)))
    args = [node_repr, edge_repr] + list(idx_t)
    if extra is not None:
        args.append(extra)
    return fn(*args)


B_TC = 5000
PER_DEG = NPER // B_TC


def _dense124_body(node_ref, sn_ref, se_ref, ws_ref, w1_ref, w2_ref, w4_ref,
                   bias_ref, y_ref, stats_ref):
    i = pl.program_id(0)
    di = i // PER_DEG
    w = jnp.where(di == 0, w1_ref[...],
                  jnp.where(di == 1, w2_ref[...], w4_ref[...]))
    y = jnp.dot(node_ref[...], ws_ref[...], preferred_element_type=jnp.float32)
    y += jnp.dot(sn_ref[...], w[:NODE], preferred_element_type=jnp.float32)
    y += jnp.dot(se_ref[...], w[NODE:], preferred_element_type=jnp.float32)
    y += bias_ref[...]
    y_ref[...] = y

    @pl.when(i == 0)
    def _():
        stats_ref[...] = jnp.zeros_like(stats_ref)

    s1 = jnp.sum(y, axis=0, keepdims=True)
    s2 = jnp.sum(y * y, axis=0, keepdims=True)
    stats_ref[...] += jnp.concatenate([s1, s2], axis=0)


def _dense8_body(node_ref, sn_ref, se_ref, ws_ref, w8_ref, bias_ref,
                 y_ref, stats_ref):
    i = pl.program_id(0)
    w = w8_ref[...]
    y = jnp.dot(node_ref[...], ws_ref[...], preferred_element_type=jnp.float32)
    y += jnp.dot(sn_ref[...], w[:NODE], preferred_element_type=jnp.float32)
    y += jnp.dot(se_ref[...], w[NODE:], preferred_element_type=jnp.float32)
    y += bias_ref[...]
    y_ref[...] = y

    @pl.when(i == 0)
    def _():
        stats_ref[...] = jnp.zeros_like(stats_ref)

    s1 = jnp.sum(y, axis=0, keepdims=True)
    s2 = jnp.sum(y * y, axis=0, keepdims=True)
    stats_ref[...] += jnp.concatenate([s1, s2], axis=0)


def _norm_body(y124_ref, y8_ref, stats124_ref, stats8_ref, out_ref):
    i = pl.program_id(0)
    s = stats124_ref[...] + stats8_ref[...]
    mean = s[0:1] * (1.0 / N)
    var = s[1:2] * (1.0 / N) - mean * mean
    inv = lax.rsqrt(var + 1e-5)
    y = jnp.where(i < 15, y124_ref[...], y8_ref[...])
    out_ref[...] = jnp.maximum((y - mean) * inv, 0.0)


def kernel(node_repr, edge_repr, node_idx_d1, edge_idx_d1, node_idx_d2,
           edge_idx_d2, node_idx_d4, edge_idx_d4, node_idx_d8, edge_idx_d8,
           W_self, W_d1, W_d2, W_d4, W_d8, bias):
    t = jnp.transpose
    idx124 = [t(node_idx_d1), t(edge_idx_d1), t(node_idx_d2),
              t(edge_idx_d2), t(node_idx_d4), t(edge_idx_d4)]
    idx8 = [t(node_idx_d8), t(edge_idx_d8)]

    sn124, se124 = _run_sc_gather(node_repr, edge_repr, idx124, (1, 2, 4))
    sn8, se8 = _run_sc_gather(node_repr, edge_repr, idx8, (8,), extra=se124)

    y124, stats124 = pl.pallas_call(
        _dense124_body,
        grid=(15,),
        in_specs=[
            pl.BlockSpec((B_TC, NODE), lambda i: (i, 0)),
            pl.BlockSpec((B_TC, NODE), lambda i: (i, 0)),
            pl.BlockSpec((B_TC, EDGE), lambda i: (i, 0)),
            pl.BlockSpec((NODE, OUT), lambda i: (0, 0)),
            pl.BlockSpec((NODE + EDGE, OUT), lambda i: (0, 0)),
            pl.BlockSpec((NODE + EDGE, OUT), lambda i: (0, 0)),
            pl.BlockSpec((NODE + EDGE, OUT), lambda i: (0, 0)),
            pl.BlockSpec((1, OUT), lambda i: (0, 0)),
        ],
        out_specs=[
            pl.BlockSpec((B_TC, OUT), lambda i: (i, 0)),
            pl.BlockSpec((2, OUT), lambda i: (0, 0)),
        ],
        out_shape=[
            jax.ShapeDtypeStruct((3 * NPER, OUT), jnp.float32),
            jax.ShapeDtypeStruct((2, OUT), jnp.float32),
        ],
    )(node_repr, sn124, se124, W_self, W_d1, W_d2, W_d4, bias)

    y8, stats8 = pl.pallas_call(
        _dense8_body,
        grid=(5,),
        in_specs=[
            pl.BlockSpec((B_TC, NODE), lambda i: (i + 15, 0)),
            pl.BlockSpec((B_TC, NODE), lambda i: (i, 0)),
            pl.BlockSpec((B_TC, EDGE), lambda i: (i, 0)),
            pl.BlockSpec((NODE, OUT), lambda i: (0, 0)),
            pl.BlockSpec((NODE + EDGE, OUT), lambda i: (0, 0)),
            pl.BlockSpec((1, OUT), lambda i: (0, 0)),
        ],
        out_specs=[
            pl.BlockSpec((B_TC, OUT), lambda i: (i, 0)),
            pl.BlockSpec((2, OUT), lambda i: (0, 0)),
        ],
        out_shape=[
            jax.ShapeDtypeStruct((NPER, OUT), jnp.float32),
            jax.ShapeDtypeStruct((2, OUT), jnp.float32),
        ],
    )(node_repr, sn8, se8, W_self, W_d8, bias)

    out = pl.pallas_call(
        _norm_body,
        grid=(N // B_TC,),
        in_specs=[
            pl.BlockSpec((B_TC, OUT), lambda i: (jnp.minimum(i, 14), 0)),
            pl.BlockSpec((B_TC, OUT), lambda i: (jnp.maximum(i - 15, 0), 0)),
            pl.BlockSpec((2, OUT), lambda i: (0, 0)),
            pl.BlockSpec((2, OUT), lambda i: (0, 0)),
        ],
        out_specs=pl.BlockSpec((B_TC, OUT), lambda i: (i, 0)),
        out_shape=jax.ShapeDtypeStruct((N, OUT), jnp.float32),
    )(y124, y8, stats124, stats8)
    return out


# speedup vs baseline: 1.5543x; 1.0120x over previous
import functools

import jax
import jax.numpy as jnp
from jax import lax
from jax.experimental import pallas as pl
from jax.experimental.pallas import tpu as pltpu
from jax.experimental.pallas import tpu_sc as plsc

N = 100000
NODE = 128
EDGE = 16
OUT = 128
E_TOT = 400000
DEGREES = (1, 2, 4, 8)
NPER = 25000

NW = 32
CHW = 784
LO_MAX = NPER - CHW
CH_D = {1: 256, 2: 112, 4: 56, 8: 32}
GBUF = 256
ABUF = 112


def _idx_slices(ch):
    out = []
    off = 0
    while off < ch:
        sz = min(128, ch - off)
        out.append((off, sz))
        off += sz
    return out


def _make_sc_body(degrees, chd):
    def body(*refs):
        it = iter(refs)
        node_hbm = next(it)
        edge_hbm = next(it)
        nidx, eidx = {}, {}
        for d in degrees:
            nidx[d] = next(it)
            eidx[d] = next(it)
        if degrees == (8,):
            next(it)
        sn_hbm = next(it)
        se_hbm = next(it)
        stg_n, stg_e = {}, {}
        for d in degrees:
            stg_n[d] = next(it)
            stg_e[d] = next(it)
        g_node = next(it)
        g_edge = next(it)
        acc_node = next(it)
        acc_edge = next(it)
        sem_gn = next(it)
        sem_ge = next(it)
        sem_sn = next(it)
        sem_se = next(it)

        wid = lax.axis_index("s") * 2 + lax.axis_index("c")
        lo = jnp.minimum(wid * CHW, LO_MAX)

        for di, d in enumerate(degrees):
            ch = chd[d]
            nch = -(-CHW // ch)
            out_off = di * NPER
            slices = _idx_slices(ch)
            n_acc = g_node if d == 1 else acc_node
            e_acc = g_edge if d == 1 else acc_edge

            def chunk_base(k, ch=ch):
                return lo + jnp.minimum(k * ch, CHW - ch)

            def stage_idx(k, p, d=d, ch=ch):
                base = chunk_base(k, ch)
                pltpu.sync_copy(nidx[d].at[pl.ds(0, d), pl.ds(base, ch)],
                                stg_n[d].at[p])
                pltpu.sync_copy(eidx[d].at[pl.ds(0, d), pl.ds(base, ch)],
                                stg_e[d].at[p])

            def fire_gathers(p, d=d, ch=ch, slices=slices):
                for j in range(d):
                    for off, sz in slices:
                        dst = pl.ds(j * ch + off, sz)
                        pltpu.async_copy(
                            node_hbm.at[stg_n[d].at[p, j, pl.ds(off, sz)]],
                            g_node.at[p, dst], sem_gn)
                        pltpu.async_copy(
                            edge_hbm.at[stg_e[d].at[p, j, pl.ds(off, sz)]],
                            g_edge.at[p, dst], sem_ge)

            def drain_gathers(p, d=d, ch=ch, slices=slices):
                for j in range(d):
                    for off, sz in slices:
                        dst = pl.ds(j * ch + off, sz)
                        pltpu.make_async_copy(node_hbm.at[pl.ds(0, sz)],
                                              g_node.at[p, dst], sem_gn).wait()
                        pltpu.make_async_copy(edge_hbm.at[pl.ds(0, sz)],
                                              g_edge.at[p, dst], sem_ge).wait()

            def drain_scatters(p, ch=ch, n_acc=n_acc, e_acc=e_acc):
                pltpu.make_async_copy(n_acc.at[p, pl.ds(0, ch)],
                                      sn_hbm.at[pl.ds(lo, ch)], sem_sn).wait()
                pltpu.make_async_copy(e_acc.at[p, pl.ds(0, ch)],
                                      se_hbm.at[pl.ds(lo, ch)], sem_se).wait()

            stage_idx(0, 0)
            fire_gathers(0)

            def chunk_body(k, _, d=d, ch=ch, nch=nch, out_off=out_off,
                           n_acc=n_acc, e_acc=e_acc, stage_idx=stage_idx,
                           fire_gathers=fire_gathers,
                           drain_gathers=drain_gathers,
                           drain_scatters=drain_scatters,
                           chunk_base=chunk_base):
                p = lax.rem(k, 2)

                @pl.when(k + 1 < nch)
                def _():
                    stage_idx(k + 1, 1 - p)

                drain_gathers(p)

                @pl.when(k + 1 < nch)
                def _():
                    if d == 1:
                        @pl.when(k >= 1)
                        def _():
                            drain_scatters(1 - p)
                    fire_gathers(1 - p)

                if d > 1:
                    @pl.when(k >= 2)
                    def _():
                        drain_scatters(p)

                    def sum_body(b, _):
                        for cseg in range(NODE // 16):
                            cs = pl.ds(cseg * 16, 16)
                            v = g_node[p, b, cs]
                            for j in range(1, d):
                                v = v + g_node[p, j * ch + b, cs]
                            acc_node[p, b, cs] = v
                        ev = g_edge[p, b, :]
                        for j in range(1, d):
                            ev = ev + g_edge[p, j * ch + b, :]
                        acc_edge[p, b, :] = ev
                        return 0
                    lax.fori_loop(0, ch, sum_body, 0, unroll=False)

                base = chunk_base(k)
                pltpu.async_copy(n_acc.at[p, pl.ds(0, ch)],
                                 sn_hbm.at[pl.ds(out_off + base, ch)], sem_sn)
                pltpu.async_copy(e_acc.at[p, pl.ds(0, ch)],
                                 se_hbm.at[pl.ds(out_off + base, ch)], sem_se)
                return 0

            lax.fori_loop(0, nch, chunk_body, 0, unroll=False)

            drain_scatters((nch - 1) % 2)
            drain_scatters((nch - 2) % 2)

    return body


def _run_sc_gather(node_repr, edge_repr, idx_t, degrees, extra=None,
                   chd=None, gbuf=GBUF, abuf=ABUF):
    chd = dict(CH_D) if chd is None else chd
    nrows = NPER * len(degrees)
    mesh = plsc.VectorSubcoreMesh(core_axis_name="c", subcore_axis_name="s")
    fn = functools.partial(
        pl.kernel,
        out_type=[
            jax.ShapeDtypeStruct((nrows, NODE), jnp.float32),
            jax.ShapeDtypeStruct((nrows, EDGE), jnp.float32),
        ],
        mesh=mesh,
        scratch_types=(
            [pltpu.VMEM((2, d, chd[d]), jnp.int32)
             for d in degrees for _ in range(2)]
            + [
                pltpu.VMEM((2, gbuf, NODE), jnp.float32),
                pltpu.VMEM((2, gbuf, EDGE), jnp.float32),
                pltpu.VMEM((2, abuf, NODE), jnp.float32),
                pltpu.VMEM((2, abuf, EDGE), jnp.float32),
                pltpu.SemaphoreType.DMA,
                pltpu.SemaphoreType.DMA,
                pltpu.SemaphoreType.DMA,
                pltpu.SemaphoreType.DMA,
            ]
        ),
        compiler_params=pltpu.CompilerParams(use_tc_tiling_on_sc=False),
    )(_make_sc_body(tuple(degrees), chd))
    args = [node_repr, edge_repr] + list(idx_t)
    if extra is not None:
        args.append(extra)
    return fn(*args)


B_TC = 5000
PER_DEG = NPER // B_TC


def _dense124_body(node_ref, sn_ref, se_ref, ws_ref, w1_ref, w2_ref, w4_ref,
                   bias_ref, y_ref, stats_ref):
    i = pl.program_id(0)
    di = i // PER_DEG
    w = jnp.where(di == 0, w1_ref[...],
                  jnp.where(di == 1, w2_ref[...], w4_ref[...]))
    y = jnp.dot(node_ref[...], ws_ref[...], preferred_element_type=jnp.float32)
    y += jnp.dot(sn_ref[...], w[:NODE], preferred_element_type=jnp.float32)
    y += jnp.dot(se_ref[...], w[NODE:], preferred_element_type=jnp.float32)
    y += bias_ref[...]
    y_ref[...] = y

    @pl.when(i == 0)
    def _():
        stats_ref[...] = jnp.zeros_like(stats_ref)

    s1 = jnp.sum(y, axis=0, keepdims=True)
    s2 = jnp.sum(y * y, axis=0, keepdims=True)
    stats_ref[...] += jnp.concatenate([s1, s2], axis=0)


def _dense8_body(node_ref, sn_ref, se_ref, ws_ref, w8_ref, bias_ref,
                 y_ref, stats_ref):
    i = pl.program_id(0)
    w = w8_ref[...]
    y = jnp.dot(node_ref[...], ws_ref[...], preferred_element_type=jnp.float32)
    y += jnp.dot(sn_ref[...], w[:NODE], preferred_element_type=jnp.float32)
    y += jnp.dot(se_ref[...], w[NODE:], preferred_element_type=jnp.float32)
    y += bias_ref[...]
    y_ref[...] = y

    @pl.when(i == 0)
    def _():
        stats_ref[...] = jnp.zeros_like(stats_ref)

    s1 = jnp.sum(y, axis=0, keepdims=True)
    s2 = jnp.sum(y * y, axis=0, keepdims=True)
    stats_ref[...] += jnp.concatenate([s1, s2], axis=0)


def _norm_body(y124_ref, y8_ref, stats124_ref, stats8_ref, out_ref):
    i = pl.program_id(0)
    s = stats124_ref[...] + stats8_ref[...]
    mean = s[0:1] * (1.0 / N)
    var = s[1:2] * (1.0 / N) - mean * mean
    inv = lax.rsqrt(var + 1e-5)
    y = jnp.where(i < 15, y124_ref[...], y8_ref[...])
    out_ref[...] = jnp.maximum((y - mean) * inv, 0.0)


def kernel(node_repr, edge_repr, node_idx_d1, edge_idx_d1, node_idx_d2,
           edge_idx_d2, node_idx_d4, edge_idx_d4, node_idx_d8, edge_idx_d8,
           W_self, W_d1, W_d2, W_d4, W_d8, bias):
    t = jnp.transpose
    idx124 = [t(node_idx_d1), t(edge_idx_d1), t(node_idx_d2),
              t(edge_idx_d2), t(node_idx_d4), t(edge_idx_d4)]
    idx8 = [t(node_idx_d8), t(edge_idx_d8)]

    sn124, se124 = _run_sc_gather(node_repr, edge_repr, idx124, (1, 2, 4))
    sn8, se8 = _run_sc_gather(node_repr, edge_repr, idx8, (8,), extra=se124,
                              chd={8: 48}, gbuf=384, abuf=48)

    y124, stats124 = pl.pallas_call(
        _dense124_body,
        grid=(15,),
        in_specs=[
            pl.BlockSpec((B_TC, NODE), lambda i: (i, 0)),
            pl.BlockSpec((B_TC, NODE), lambda i: (i, 0)),
            pl.BlockSpec((B_TC, EDGE), lambda i: (i, 0)),
            pl.BlockSpec((NODE, OUT), lambda i: (0, 0)),
            pl.BlockSpec((NODE + EDGE, OUT), lambda i: (0, 0)),
            pl.BlockSpec((NODE + EDGE, OUT), lambda i: (0, 0)),
            pl.BlockSpec((NODE + EDGE, OUT), lambda i: (0, 0)),
            pl.BlockSpec((1, OUT), lambda i: (0, 0)),
        ],
        out_specs=[
            pl.BlockSpec((B_TC, OUT), lambda i: (i, 0)),
            pl.BlockSpec((2, OUT), lambda i: (0, 0)),
        ],
        out_shape=[
            jax.ShapeDtypeStruct((3 * NPER, OUT), jnp.float32),
            jax.ShapeDtypeStruct((2, OUT), jnp.float32),
        ],
    )(node_repr, sn124, se124, W_self, W_d1, W_d2, W_d4, bias)

    y8, stats8 = pl.pallas_call(
        _dense8_body,
        grid=(5,),
        in_specs=[
            pl.BlockSpec((B_TC, NODE), lambda i: (i + 15, 0)),
            pl.BlockSpec((B_TC, NODE), lambda i: (i, 0)),
            pl.BlockSpec((B_TC, EDGE), lambda i: (i, 0)),
            pl.BlockSpec((NODE, OUT), lambda i: (0, 0)),
            pl.BlockSpec((NODE + EDGE, OUT), lambda i: (0, 0)),
            pl.BlockSpec((1, OUT), lambda i: (0, 0)),
        ],
        out_specs=[
            pl.BlockSpec((B_TC, OUT), lambda i: (i, 0)),
            pl.BlockSpec((2, OUT), lambda i: (0, 0)),
        ],
        out_shape=[
            jax.ShapeDtypeStruct((NPER, OUT), jnp.float32),
            jax.ShapeDtypeStruct((2, OUT), jnp.float32),
        ],
    )(node_repr, sn8, se8, W_self, W_d8, bias)

    out = pl.pallas_call(
        _norm_body,
        grid=(N // B_TC,),
        in_specs=[
            pl.BlockSpec((B_TC, OUT), lambda i: (jnp.minimum(i, 14), 0)),
            pl.BlockSpec((B_TC, OUT), lambda i: (jnp.maximum(i - 15, 0), 0)),
            pl.BlockSpec((2, OUT), lambda i: (0, 0)),
            pl.BlockSpec((2, OUT), lambda i: (0, 0)),
        ],
        out_specs=pl.BlockSpec((B_TC, OUT), lambda i: (i, 0)),
        out_shape=jax.ShapeDtypeStruct((N, OUT), jnp.float32),
    )(y124, y8, stats124, stats8)
    return out


# speedup vs baseline: 1.9328x; 1.2435x over previous
import functools

import jax
import jax.numpy as jnp
from jax import lax
from jax.experimental import pallas as pl
from jax.experimental.pallas import tpu as pltpu
from jax.experimental.pallas import tpu_sc as plsc

N = 100000
NODE = 128
EDGE = 16
OUT = 128
E_TOT = 400000
DEGREES = (1, 2, 4, 8)
NPER = 25000

NW = 32
CHW = 784
LO_MAX = NPER - CHW
CH_D = {1: 256, 2: 112, 4: 56, 8: 32}
GBUF = 256
ABUF = 112


def _idx_slices(ch):
    out = []
    off = 0
    while off < ch:
        sz = min(128, ch - off)
        out.append((off, sz))
        off += sz
    return out


def _make_sc_body(degrees, chd, ncols, has_extra):
    def body(*refs):
        it = iter(refs)
        table_hbm = next(it)
        idx = {}
        for d in degrees:
            idx[d] = next(it)
        if has_extra:
            next(it)
        out_hbm = next(it)
        stg = {}
        for d in degrees:
            stg[d] = next(it)
        g_buf = next(it)
        acc = next(it)
        sem_g = next(it)
        sem_s = next(it)

        wid = lax.axis_index("s") * 2 + lax.axis_index("c")
        lo = jnp.minimum(wid * CHW, LO_MAX)

        for di, d in enumerate(degrees):
            ch = chd[d]
            nch = -(-CHW // ch)
            out_off = di * NPER
            slices = _idx_slices(ch)
            o_acc = g_buf if d == 1 else acc

            def chunk_base(k, ch=ch):
                return lo + jnp.minimum(k * ch, CHW - ch)

            def stage_idx(k, p, d=d, ch=ch):
                base = chunk_base(k, ch)
                pltpu.sync_copy(idx[d].at[pl.ds(0, d), pl.ds(base, ch)],
                                stg[d].at[p])

            def fire_gathers(p, d=d, ch=ch, slices=slices):
                for j in range(d):
                    for off, sz in slices:
                        dst = pl.ds(j * ch + off, sz)
                        pltpu.async_copy(
                            table_hbm.at[stg[d].at[p, j, pl.ds(off, sz)]],
                            g_buf.at[p, dst], sem_g)

            def drain_gathers(p, d=d, ch=ch, slices=slices):
                for j in range(d):
                    for off, sz in slices:
                        dst = pl.ds(j * ch + off, sz)
                        pltpu.make_async_copy(table_hbm.at[pl.ds(0, sz)],
                                              g_buf.at[p, dst], sem_g).wait()

            def drain_scatters(p, ch=ch, o_acc=o_acc):
                pltpu.make_async_copy(o_acc.at[p, pl.ds(0, ch)],
                                      out_hbm.at[pl.ds(lo, ch)], sem_s).wait()

            stage_idx(0, 0)
            fire_gathers(0)

            def chunk_body(k, _, d=d, ch=ch, nch=nch, out_off=out_off,
                           o_acc=o_acc, stage_idx=stage_idx,
                           fire_gathers=fire_gathers,
                           drain_gathers=drain_gathers,
                           drain_scatters=drain_scatters,
                           chunk_base=chunk_base):
                p = lax.rem(k, 2)

                @pl.when(k + 1 < nch)
                def _():
                    stage_idx(k + 1, 1 - p)

                drain_gathers(p)

                @pl.when(k + 1 < nch)
                def _():
                    if d == 1:
                        @pl.when(k >= 1)
                        def _():
                            drain_scatters(1 - p)
                    fire_gathers(1 - p)

                if d > 1:
                    @pl.when(k >= 2)
                    def _():
                        drain_scatters(p)

                    def sum_body(b, _):
                        for cseg in range(ncols // 16):
                            cs = pl.ds(cseg * 16, 16)
                            v = g_buf[p, b, cs]
                            for j in range(1, d):
                                v = v + g_buf[p, j * ch + b, cs]
                            acc[p, b, cs] = v
                        return 0
                    lax.fori_loop(0, ch, sum_body, 0, unroll=False)

                base = chunk_base(k)
                pltpu.async_copy(o_acc.at[p, pl.ds(0, ch)],
                                 out_hbm.at[pl.ds(out_off + base, ch)], sem_s)
                return 0

            lax.fori_loop(0, nch, chunk_body, 0, unroll=False)

            drain_scatters((nch - 1) % 2)
            drain_scatters((nch - 2) % 2)

    return body


def _run_sc_gather(table, idx_t, degrees, ncols, chd, gbuf, abuf,
                   extra=None):
    nrows = NPER * len(degrees)
    mesh = plsc.VectorSubcoreMesh(core_axis_name="c", subcore_axis_name="s")
    fn = functools.partial(
        pl.kernel,
        out_type=jax.ShapeDtypeStruct((nrows, ncols), jnp.float32),
        mesh=mesh,
        scratch_types=(
            [pltpu.VMEM((2, d, chd[d]), jnp.int32) for d in degrees]
            + [
                pltpu.VMEM((2, gbuf, ncols), jnp.float32),
                pltpu.VMEM((2, abuf, ncols), jnp.float32),
                pltpu.SemaphoreType.DMA,
                pltpu.SemaphoreType.DMA,
            ]
        ),
        compiler_params=pltpu.CompilerParams(use_tc_tiling_on_sc=False),
    )(_make_sc_body(tuple(degrees), chd, ncols, extra is not None))
    args = [table] + list(idx_t)
    if extra is not None:
        args.append(extra)
    return fn(*args)


B_TC = 5000
PER_DEG = NPER // B_TC


def _dense124_body(node_ref, sn_ref, se_ref, ws_ref, w1_ref, w2_ref, w4_ref,
                   bias_ref, y_ref, stats_ref):
    i = pl.program_id(0)
    di = i // PER_DEG
    w = jnp.where(di == 0, w1_ref[...],
                  jnp.where(di == 1, w2_ref[...], w4_ref[...]))
    y = jnp.dot(node_ref[...], ws_ref[...], preferred_element_type=jnp.float32)
    y += jnp.dot(sn_ref[...], w[:NODE], preferred_element_type=jnp.float32)
    y += jnp.dot(se_ref[...], w[NODE:], preferred_element_type=jnp.float32)
    y += bias_ref[...]
    y_ref[...] = y

    @pl.when(i == 0)
    def _():
        stats_ref[...] = jnp.zeros_like(stats_ref)

    s1 = jnp.sum(y, axis=0, keepdims=True)
    s2 = jnp.sum(y * y, axis=0, keepdims=True)
    stats_ref[...] += jnp.concatenate([s1, s2], axis=0)


def _dense8_body(node_ref, sn_ref, se_ref, ws_ref, w8_ref, bias_ref,
                 y_ref, stats_ref):
    i = pl.program_id(0)
    w = w8_ref[...]
    y = jnp.dot(node_ref[...], ws_ref[...], preferred_element_type=jnp.float32)
    y += jnp.dot(sn_ref[...], w[:NODE], preferred_element_type=jnp.float32)
    y += jnp.dot(se_ref[...], w[NODE:], preferred_element_type=jnp.float32)
    y += bias_ref[...]
    y_ref[...] = y

    @pl.when(i == 0)
    def _():
        stats_ref[...] = jnp.zeros_like(stats_ref)

    s1 = jnp.sum(y, axis=0, keepdims=True)
    s2 = jnp.sum(y * y, axis=0, keepdims=True)
    stats_ref[...] += jnp.concatenate([s1, s2], axis=0)


def _norm_body(y124_ref, y8_ref, stats124_ref, stats8_ref, out_ref):
    i = pl.program_id(0)
    s = stats124_ref[...] + stats8_ref[...]
    mean = s[0:1] * (1.0 / N)
    var = s[1:2] * (1.0 / N) - mean * mean
    inv = lax.rsqrt(var + 1e-5)
    y = jnp.where(i < 15, y124_ref[...], y8_ref[...])
    out_ref[...] = jnp.maximum((y - mean) * inv, 0.0)


def kernel(node_repr, edge_repr, node_idx_d1, edge_idx_d1, node_idx_d2,
           edge_idx_d2, node_idx_d4, edge_idx_d4, node_idx_d8, edge_idx_d8,
           W_self, W_d1, W_d2, W_d4, W_d8, bias):
    t = jnp.transpose
    sn124 = _run_sc_gather(
        node_repr, [t(node_idx_d1), t(node_idx_d2), t(node_idx_d4)],
        (1, 2, 4), NODE, chd={1: 256, 2: 112, 4: 56}, gbuf=256, abuf=112)
    se = _run_sc_gather(
        edge_repr, [t(edge_idx_d1), t(edge_idx_d2), t(edge_idx_d4),
                    t(edge_idx_d8)],
        (1, 2, 4, 8), EDGE, chd={1: 512, 2: 512, 4: 256, 8: 128},
        gbuf=1024, abuf=512, extra=sn124)
    sn8 = _run_sc_gather(
        node_repr, [t(node_idx_d8)], (8,), NODE,
        chd={8: 48}, gbuf=384, abuf=48, extra=se)

    y124, stats124 = pl.pallas_call(
        _dense124_body,
        grid=(15,),
        in_specs=[
            pl.BlockSpec((B_TC, NODE), lambda i: (i, 0)),
            pl.BlockSpec((B_TC, NODE), lambda i: (i, 0)),
            pl.BlockSpec((B_TC, EDGE), lambda i: (i, 0)),
            pl.BlockSpec((NODE, OUT), lambda i: (0, 0)),
            pl.BlockSpec((NODE + EDGE, OUT), lambda i: (0, 0)),
            pl.BlockSpec((NODE + EDGE, OUT), lambda i: (0, 0)),
            pl.BlockSpec((NODE + EDGE, OUT), lambda i: (0, 0)),
            pl.BlockSpec((1, OUT), lambda i: (0, 0)),
        ],
        out_specs=[
            pl.BlockSpec((B_TC, OUT), lambda i: (i, 0)),
            pl.BlockSpec((2, OUT), lambda i: (0, 0)),
        ],
        out_shape=[
            jax.ShapeDtypeStruct((3 * NPER, OUT), jnp.float32),
            jax.ShapeDtypeStruct((2, OUT), jnp.float32),
        ],
    )(node_repr, sn124, se, W_self, W_d1, W_d2, W_d4, bias)

    y8, stats8 = pl.pallas_call(
        _dense8_body,
        grid=(5,),
        in_specs=[
            pl.BlockSpec((B_TC, NODE), lambda i: (i + 15, 0)),
            pl.BlockSpec((B_TC, NODE), lambda i: (i, 0)),
            pl.BlockSpec((B_TC, EDGE), lambda i: (i + 15, 0)),
            pl.BlockSpec((NODE, OUT), lambda i: (0, 0)),
            pl.BlockSpec((NODE + EDGE, OUT), lambda i: (0, 0)),
            pl.BlockSpec((1, OUT), lambda i: (0, 0)),
        ],
        out_specs=[
            pl.BlockSpec((B_TC, OUT), lambda i: (i, 0)),
            pl.BlockSpec((2, OUT), lambda i: (0, 0)),
        ],
        out_shape=[
            jax.ShapeDtypeStruct((NPER, OUT), jnp.float32),
            jax.ShapeDtypeStruct((2, OUT), jnp.float32),
        ],
    )(node_repr, sn8, se, W_self, W_d8, bias)

    out = pl.pallas_call(
        _norm_body,
        grid=(N // B_TC,),
        in_specs=[
            pl.BlockSpec((B_TC, OUT), lambda i: (jnp.minimum(i, 14), 0)),
            pl.BlockSpec((B_TC, OUT), lambda i: (jnp.maximum(i - 15, 0), 0)),
            pl.BlockSpec((2, OUT), lambda i: (0, 0)),
            pl.BlockSpec((2, OUT), lambda i: (0, 0)),
        ],
        out_specs=pl.BlockSpec((B_TC, OUT), lambda i: (i, 0)),
        out_shape=jax.ShapeDtypeStruct((N, OUT), jnp.float32),
    )(y124, y8, stats124, stats8)
    return out


# speedup vs baseline: 1.9388x; 1.0031x over previous
import functools

import jax
import jax.numpy as jnp
from jax import lax
from jax.experimental import pallas as pl
from jax.experimental.pallas import tpu as pltpu
from jax.experimental.pallas import tpu_sc as plsc

N = 100000
NODE = 128
EDGE = 16
OUT = 128
E_TOT = 400000
DEGREES = (1, 2, 4, 8)
NPER = 25000

NW = 32
CHW = 784
LO_MAX = NPER - CHW
CH_D = {1: 256, 2: 112, 4: 56, 8: 32}
GBUF = 256
ABUF = 112


def _idx_slices(ch):
    out = []
    off = 0
    while off < ch:
        sz = min(128, ch - off)
        out.append((off, sz))
        off += sz
    return out


def _make_sc_body(degrees, chd, ncols, has_extra, out_cols):
    def body(*refs):
        it = iter(refs)
        table_hbm = next(it)
        idx = {}
        for d in degrees:
            idx[d] = next(it)
        if has_extra:
            next(it)
        out_hbm = next(it)
        stg = {}
        for d in degrees:
            stg[d] = next(it)
        g_buf = next(it)
        acc = next(it)
        sem_g = next(it)
        sem_s = next(it)

        wid = lax.axis_index("s") * 2 + lax.axis_index("c")
        lo = jnp.minimum(wid * CHW, LO_MAX)

        for di, d in enumerate(degrees):
            ch = chd[d]
            nch = -(-CHW // ch)
            out_off = di * NPER
            slices = _idx_slices(ch)
            o_acc = g_buf if d == 1 else acc

            def chunk_base(k, ch=ch):
                return lo + jnp.minimum(k * ch, CHW - ch)

            def stage_idx(k, p, d=d, ch=ch):
                base = chunk_base(k, ch)
                pltpu.sync_copy(idx[d].at[pl.ds(0, d), pl.ds(base, ch)],
                                stg[d].at[p])

            def fire_gathers(p, d=d, ch=ch, slices=slices):
                for j in range(d):
                    for off, sz in slices:
                        dst = pl.ds(j * ch + off, sz)
                        pltpu.async_copy(
                            table_hbm.at[stg[d].at[p, j, pl.ds(off, sz)]],
                            g_buf.at[p, dst], sem_g)

            def drain_gathers(p, d=d, ch=ch, slices=slices):
                for j in range(d):
                    for off, sz in slices:
                        dst = pl.ds(j * ch + off, sz)
                        pltpu.make_async_copy(table_hbm.at[pl.ds(0, sz)],
                                              g_buf.at[p, dst], sem_g).wait()

            def out_slice(r0, ch):
                if out_cols == ncols:
                    return out_hbm.at[pl.ds(r0, ch)]
                return out_hbm.at[pl.ds(r0, ch), pl.ds(0, ncols)]

            def drain_scatters(p, ch=ch, o_acc=o_acc):
                pltpu.make_async_copy(o_acc.at[p, pl.ds(0, ch)],
                                      out_slice(lo, ch), sem_s).wait()

            stage_idx(0, 0)
            fire_gathers(0)

            def chunk_body(k, _, d=d, ch=ch, nch=nch, out_off=out_off,
                           o_acc=o_acc, stage_idx=stage_idx,
                           fire_gathers=fire_gathers,
                           drain_gathers=drain_gathers,
                           drain_scatters=drain_scatters,
                           chunk_base=chunk_base):
                p = lax.rem(k, 2)

                @pl.when(k + 1 < nch)
                def _():
                    stage_idx(k + 1, 1 - p)

                drain_gathers(p)

                @pl.when(k + 1 < nch)
                def _():
                    if d == 1:
                        @pl.when(k >= 1)
                        def _():
                            drain_scatters(1 - p)
                    fire_gathers(1 - p)

                if d > 1:
                    @pl.when(k >= 2)
                    def _():
                        drain_scatters(p)

                    def sum_body(b, _):
                        for cseg in range(ncols // 16):
                            cs = pl.ds(cseg * 16, 16)
                            v = g_buf[p, b, cs]
                            for j in range(1, d):
                                v = v + g_buf[p, j * ch + b, cs]
                            acc[p, b, cs] = v
                        return 0
                    lax.fori_loop(0, ch, sum_body, 0, unroll=False)

                base = chunk_base(k)
                pltpu.async_copy(o_acc.at[p, pl.ds(0, ch)],
                                 out_slice(out_off + base, ch), sem_s)
                return 0

            lax.fori_loop(0, nch, chunk_body, 0, unroll=False)

            drain_scatters((nch - 1) % 2)
            drain_scatters((nch - 2) % 2)

    return body


def _run_sc_gather(table, idx_t, degrees, ncols, chd, gbuf, abuf,
                   extra=None, out_cols=None):
    out_cols = ncols if out_cols is None else out_cols
    nrows = NPER * len(degrees)
    mesh = plsc.VectorSubcoreMesh(core_axis_name="c", subcore_axis_name="s")
    fn = functools.partial(
        pl.kernel,
        out_type=jax.ShapeDtypeStruct((nrows, out_cols), jnp.float32),
        mesh=mesh,
        scratch_types=(
            [pltpu.VMEM((2, d, chd[d]), jnp.int32) for d in degrees]
            + [
                pltpu.VMEM((2, gbuf, ncols), jnp.float32),
                pltpu.VMEM((2, abuf, ncols), jnp.float32),
                pltpu.SemaphoreType.DMA,
                pltpu.SemaphoreType.DMA,
            ]
        ),
        compiler_params=pltpu.CompilerParams(use_tc_tiling_on_sc=False),
    )(_make_sc_body(tuple(degrees), chd, ncols, extra is not None, out_cols))
    args = [table] + list(idx_t)
    if extra is not None:
        args.append(extra)
    return fn(*args)


B_TC = 5000
PER_DEG = NPER // B_TC


def _dense124_body(node_ref, sn_ref, se_ref, ws_ref, w1_ref, w2_ref, w4_ref,
                   bias_ref, y_ref, stats_ref):
    i = pl.program_id(0)
    di = i // PER_DEG
    w = jnp.where(di == 0, w1_ref[...],
                  jnp.where(di == 1, w2_ref[...], w4_ref[...]))
    y = jnp.dot(node_ref[...], ws_ref[...], preferred_element_type=jnp.float32)
    y += jnp.dot(sn_ref[...], w[:NODE], preferred_element_type=jnp.float32)
    y += jnp.dot(se_ref[...][:, :EDGE], w[NODE:],
                 preferred_element_type=jnp.float32)
    y += bias_ref[...]
    y_ref[...] = y

    @pl.when(i == 0)
    def _():
        stats_ref[...] = jnp.zeros_like(stats_ref)

    s1 = jnp.sum(y, axis=0, keepdims=True)
    s2 = jnp.sum(y * y, axis=0, keepdims=True)
    stats_ref[...] += jnp.concatenate([s1, s2], axis=0)


def _dense8_body(node_ref, sn_ref, se_ref, ws_ref, w8_ref, bias_ref,
                 y_ref, stats_ref):
    i = pl.program_id(0)
    w = w8_ref[...]
    y = jnp.dot(node_ref[...], ws_ref[...], preferred_element_type=jnp.float32)
    y += jnp.dot(sn_ref[...], w[:NODE], preferred_element_type=jnp.float32)
    y += jnp.dot(se_ref[...][:, :EDGE], w[NODE:],
                 preferred_element_type=jnp.float32)
    y += bias_ref[...]
    y_ref[...] = y

    @pl.when(i == 0)
    def _():
        stats_ref[...] = jnp.zeros_like(stats_ref)

    s1 = jnp.sum(y, axis=0, keepdims=True)
    s2 = jnp.sum(y * y, axis=0, keepdims=True)
    stats_ref[...] += jnp.concatenate([s1, s2], axis=0)


def _norm_body(y124_ref, y8_ref, stats124_ref, stats8_ref, out_ref):
    i = pl.program_id(0)
    s = stats124_ref[...] + stats8_ref[...]
    mean = s[0:1] * (1.0 / N)
    var = s[1:2] * (1.0 / N) - mean * mean
    inv = lax.rsqrt(var + 1e-5)
    y = jnp.where(i < 15, y124_ref[...], y8_ref[...])
    out_ref[...] = jnp.maximum((y - mean) * inv, 0.0)


def kernel(node_repr, edge_repr, node_idx_d1, edge_idx_d1, node_idx_d2,
           edge_idx_d2, node_idx_d4, edge_idx_d4, node_idx_d8, edge_idx_d8,
           W_self, W_d1, W_d2, W_d4, W_d8, bias):
    t = jnp.transpose
    sn124 = _run_sc_gather(
        node_repr, [t(node_idx_d1), t(node_idx_d2), t(node_idx_d4)],
        (1, 2, 4), NODE, chd={1: 256, 2: 112, 4: 56}, gbuf=256, abuf=112)
    se = _run_sc_gather(
        edge_repr, [t(edge_idx_d1), t(edge_idx_d2), t(edge_idx_d4),
                    t(edge_idx_d8)],
        (1, 2, 4, 8), EDGE, chd={1: 512, 2: 512, 4: 256, 8: 128},
        gbuf=1024, abuf=512, extra=sn124, out_cols=OUT)
    sn8 = _run_sc_gather(
        node_repr, [t(node_idx_d8)], (8,), NODE,
        chd={8: 48}, gbuf=384, abuf=48, extra=se)

    y124, stats124 = pl.pallas_call(
        _dense124_body,
        grid=(15,),
        in_specs=[
            pl.BlockSpec((B_TC, NODE), lambda i: (i, 0)),
            pl.BlockSpec((B_TC, NODE), lambda i: (i, 0)),
            pl.BlockSpec((B_TC, OUT), lambda i: (i, 0)),
            pl.BlockSpec((NODE, OUT), lambda i: (0, 0)),
            pl.BlockSpec((NODE + EDGE, OUT), lambda i: (0, 0)),
            pl.BlockSpec((NODE + EDGE, OUT), lambda i: (0, 0)),
            pl.BlockSpec((NODE + EDGE, OUT), lambda i: (0, 0)),
            pl.BlockSpec((1, OUT), lambda i: (0, 0)),
        ],
        out_specs=[
            pl.BlockSpec((B_TC, OUT), lambda i: (i, 0)),
            pl.BlockSpec((2, OUT), lambda i: (0, 0)),
        ],
        out_shape=[
            jax.ShapeDtypeStruct((3 * NPER, OUT), jnp.float32),
            jax.ShapeDtypeStruct((2, OUT), jnp.float32),
        ],
    )(node_repr, sn124, se, W_self, W_d1, W_d2, W_d4, bias)

    y8, stats8 = pl.pallas_call(
        _dense8_body,
        grid=(5,),
        in_specs=[
            pl.BlockSpec((B_TC, NODE), lambda i: (i + 15, 0)),
            pl.BlockSpec((B_TC, NODE), lambda i: (i, 0)),
            pl.BlockSpec((B_TC, OUT), lambda i: (i + 15, 0)),
            pl.BlockSpec((NODE, OUT), lambda i: (0, 0)),
            pl.BlockSpec((NODE + EDGE, OUT), lambda i: (0, 0)),
            pl.BlockSpec((1, OUT), lambda i: (0, 0)),
        ],
        out_specs=[
            pl.BlockSpec((B_TC, OUT), lambda i: (i, 0)),
            pl.BlockSpec((2, OUT), lambda i: (0, 0)),
        ],
        out_shape=[
            jax.ShapeDtypeStruct((NPER, OUT), jnp.float32),
            jax.ShapeDtypeStruct((2, OUT), jnp.float32),
        ],
    )(node_repr, sn8, se, W_self, W_d8, bias)

    out = pl.pallas_call(
        _norm_body,
        grid=(N // B_TC,),
        in_specs=[
            pl.BlockSpec((B_TC, OUT), lambda i: (jnp.minimum(i, 14), 0)),
            pl.BlockSpec((B_TC, OUT), lambda i: (jnp.maximum(i - 15, 0), 0)),
            pl.BlockSpec((2, OUT), lambda i: (0, 0)),
            pl.BlockSpec((2, OUT), lambda i: (0, 0)),
        ],
        out_specs=pl.BlockSpec((B_TC, OUT), lambda i: (i, 0)),
        out_shape=jax.ShapeDtypeStruct((N, OUT), jnp.float32),
    )(y124, y8, stats124, stats8)
    return out
